# baseline (speedup 1.0000x reference)
import functools

import jax
import jax.numpy as jnp
from jax.experimental import pallas as pl
from jax.experimental.pallas import tpu as pltpu

_BM = 2048
_BN = 1024
_NEG = -1e30


def _koleo_body(n, eps, xi_ref, xj_ref, out_ref, xi_bf_ref, max_ref):
    i = pl.program_id(0)
    j = pl.program_id(1)
    ncols = pl.num_programs(1)

    @pl.when(j == 0)
    def _():
        xi_bf_ref[...] = (-2.0 * xi_ref[...]).astype(jnp.bfloat16)

    xj_blk = xj_ref[...]
    ones = jnp.ones((1, xj_blk.shape[1]), jnp.float32)
    b2 = jax.lax.dot_general(
        ones, xj_blk * xj_blk, (((1,), (1,)), ((), ())),
        preferred_element_type=jnp.float32)

    s = jax.lax.dot_general(
        xi_bf_ref[...], xj_blk.astype(jnp.bfloat16),
        (((1,), (1,)), ((), ())),
        preferred_element_type=jnp.float32)
    score = s + b2

    rows = i * _BM + jax.lax.broadcasted_iota(jnp.int32, (_BM, _BN), 0)
    cols = j * _BN + jax.lax.broadcasted_iota(jnp.int32, (_BM, _BN), 1)
    score = jnp.where(rows == cols, _NEG, score)

    m = jnp.max(score, axis=1, keepdims=True)

    @pl.when(j == 0)
    def _():
        max_ref[...] = m

    @pl.when(j > 0)
    def _():
        max_ref[...] = jnp.maximum(max_ref[...], m)

    @pl.when((i == 0) & (j == 0))
    def _():
        out_ref[...] = jnp.zeros((1, 1), jnp.float32)

    @pl.when(j == ncols - 1)
    def _():
        xi_blk = xi_ref[...]
        a2 = jnp.sum(xi_blk * xi_blk, axis=1, keepdims=True)
        d2 = a2 + max_ref[...]
        lg = jnp.log(1.0 / (d2 * 0.5 + 1.0) ** 2 + eps)
        out_ref[...] += jnp.sum(lg, keepdims=True)


def kernel(xi, xj):
    eps = 1e-08
    n, k = xi.shape

    out = pl.pallas_call(
        functools.partial(_koleo_body, n, eps),
        grid=(n // _BM, n // _BN),
        in_specs=[
            pl.BlockSpec((_BM, k), lambda i, j: (i, 0)),
            pl.BlockSpec((_BN, k), lambda i, j: (j, 0)),
        ],
        out_specs=pl.BlockSpec((1, 1), lambda i, j: (0, 0)),
        out_shape=jax.ShapeDtypeStruct((1, 1), jnp.float32),
        scratch_shapes=[
            pltpu.VMEM((_BM, k), jnp.bfloat16),
            pltpu.VMEM((_BM, 1), jnp.float32),
        ],
        compiler_params=pltpu.CompilerParams(
            dimension_semantics=("arbitrary", "arbitrary")),
    )(xi, xj)
    return out[0, 0] / n
